# Initial kernel scaffold; baseline (speedup 1.0000x reference)
import jax
import jax.numpy as jnp
from jax.experimental import pallas as pl


def kernel(token_ids, weight):
    raise NotImplementedError("write your pallas kernel here")


# speedup vs baseline: 1.1031x; 1.1031x over previous
import functools

import jax
import jax.numpy as jnp
from jax import lax
from jax.experimental import pallas as pl
from jax.experimental.pallas import tpu as pltpu
from jax.experimental.pallas import tpu_sc as plsc

NUM_TOKENS = 16384 * 50
DIM = 32
_info = plsc.get_sparse_core_info()
NC, NS = _info.num_cores, _info.num_subcores
NW = NC * NS
B_PER_W = NUM_TOKENS // NW
CHUNK = 1600
N_CHUNKS = B_PER_W // CHUNK

_mesh = plsc.VectorSubcoreMesh(core_axis_name="c", subcore_axis_name="s")


@functools.partial(
    pl.kernel,
    mesh=_mesh,
    out_type=jax.ShapeDtypeStruct((NUM_TOKENS, DIM), jnp.float32),
    scratch_types=[
        pltpu.VMEM((CHUNK,), jnp.int32),
        pltpu.VMEM((CHUNK, DIM), jnp.float32),
        pltpu.SemaphoreType.DMA,
    ],
    compiler_params=pltpu.CompilerParams(use_tc_tiling_on_sc=False),
)
def _gather_kernel(tok_hbm, table_hbm, out_hbm, idx_v, rows_v, sem):
    wid = lax.axis_index("s") * NC + lax.axis_index("c")
    base = wid * B_PER_W

    def chunk_body(i, carry):
        off = base + i * CHUNK
        pltpu.sync_copy(tok_hbm.at[pl.ds(off, CHUNK)], idx_v)
        pltpu.async_copy(table_hbm.at[idx_v], rows_v, sem).wait()
        pltpu.sync_copy(rows_v, out_hbm.at[pl.ds(off, CHUNK)])
        return carry

    lax.fori_loop(0, N_CHUNKS, chunk_body, 0)


def kernel(token_ids, weight):
    flat = token_ids.reshape(-1).astype(jnp.int32)
    out = _gather_kernel(flat, weight)
    return out.reshape(token_ids.shape + (DIM,))


# speedup vs baseline: 1.1096x; 1.0059x over previous
import functools

import jax
import jax.numpy as jnp
from jax import lax
from jax.experimental import pallas as pl
from jax.experimental.pallas import tpu as pltpu
from jax.experimental.pallas import tpu_sc as plsc

NUM_TOKENS = 16384 * 50
DIM = 32
_info = plsc.get_sparse_core_info()
NC, NS = _info.num_cores, _info.num_subcores
NW = NC * NS
B_PER_W = NUM_TOKENS // NW
CHUNK = 1600
N_CHUNKS = B_PER_W // CHUNK

_mesh = plsc.VectorSubcoreMesh(core_axis_name="c", subcore_axis_name="s")


@functools.partial(
    pl.kernel,
    mesh=_mesh,
    out_type=jax.ShapeDtypeStruct((NUM_TOKENS, DIM), jnp.float32),
    scratch_types=[
        pltpu.VMEM((CHUNK,), jnp.int32),
        pltpu.VMEM((CHUNK,), jnp.int32),
        pltpu.VMEM((CHUNK, DIM), jnp.float32),
        pltpu.VMEM((CHUNK, DIM), jnp.float32),
        pltpu.SemaphoreType.DMA,
        pltpu.SemaphoreType.DMA,
        pltpu.SemaphoreType.DMA,
        pltpu.SemaphoreType.DMA,
        pltpu.SemaphoreType.DMA,
    ],
    compiler_params=pltpu.CompilerParams(use_tc_tiling_on_sc=False),
)
def _gather_kernel(tok_hbm, table_hbm, out_hbm,
                   idx0, idx1, rows0, rows1,
                   sem_i0, sem_i1, sem_g, sem_s0, sem_s1):
    wid = lax.axis_index("s") * NC + lax.axis_index("c")
    base = wid * B_PER_W
    idx_bufs = (idx0, idx1)
    rows_bufs = (rows0, rows1)
    sem_idx = (sem_i0, sem_i1)
    sem_st = (sem_s0, sem_s1)

    pltpu.async_copy(tok_hbm.at[pl.ds(base, CHUNK)], idx0, sem_i0)
    pltpu.async_copy(tok_hbm.at[pl.ds(base + CHUNK, CHUNK)], idx1, sem_i1)

    def pair_body(p, carry):
        for b in range(2):
            g = 2 * p + b
            off = base + g * CHUNK
            pltpu.make_async_copy(
                tok_hbm.at[pl.ds(off, CHUNK)], idx_bufs[b], sem_idx[b]
            ).wait()
            @pl.when(g >= 2)
            def _wait_prev_store():
                pltpu.make_async_copy(
                    rows_bufs[b],
                    out_hbm.at[pl.ds(off - 2 * CHUNK, CHUNK)],
                    sem_st[b],
                ).wait()

            pltpu.async_copy(
                table_hbm.at[idx_bufs[b]], rows_bufs[b], sem_g
            ).wait()
            pltpu.async_copy(
                rows_bufs[b], out_hbm.at[pl.ds(off, CHUNK)], sem_st[b]
            )

            @pl.when(g + 2 < N_CHUNKS)
            def _prefetch_idx():
                pltpu.async_copy(
                    tok_hbm.at[pl.ds(off + 2 * CHUNK, CHUNK)],
                    idx_bufs[b],
                    sem_idx[b],
                )

        return carry

    lax.fori_loop(0, N_CHUNKS // 2, pair_body, 0)

    for b in range(2):
        g = N_CHUNKS - 2 + b
        pltpu.make_async_copy(
            rows_bufs[b],
            out_hbm.at[pl.ds(base + g * CHUNK, CHUNK)],
            sem_st[b],
        ).wait()


def kernel(token_ids, weight):
    flat = token_ids.reshape(-1).astype(jnp.int32)
    out = _gather_kernel(flat, weight)
    return out.reshape(token_ids.shape + (DIM,))


# speedup vs baseline: 1.4804x; 1.3342x over previous
import functools

import jax
import jax.numpy as jnp
from jax import lax
from jax.experimental import pallas as pl
from jax.experimental.pallas import tpu as pltpu
from jax.experimental.pallas import tpu_sc as plsc

B = 16384
S = 50
DIM = 32
_info = plsc.get_sparse_core_info()
NC, NS = _info.num_cores, _info.num_subcores
NW = NC * NS
B_W = B // NW
BH_W = B_W // 128

_mesh = plsc.VectorSubcoreMesh(core_axis_name="c", subcore_axis_name="s")


@functools.partial(
    pl.kernel,
    mesh=_mesh,
    out_type=jax.ShapeDtypeStruct((S, DIM // 8, B // 128, 8 * 128), jnp.float32),
    scratch_types=[
        pltpu.VMEM((B_W,), jnp.int32),
        pltpu.VMEM((B_W, DIM), jnp.float32),
        pltpu.VMEM((DIM // 8, BH_W, 8 * 128), jnp.float32),
        pltpu.SemaphoreType.DMA,
    ],
    compiler_params=pltpu.CompilerParams(
        use_tc_tiling_on_sc=False, needs_layout_passes=False
    ),
)
def _gather_kernel(tok_hbm, table_hbm, out_hbm, idx_v, rows_v, trans_v, sem):
    wid = lax.axis_index("s") * NC + lax.axis_index("c")
    b0 = wid * B_W
    bh0 = wid * BH_W
    lane = lax.iota(jnp.int32, 16)

    def s_body(s, carry):
        pltpu.sync_copy(tok_hbm.at[s, pl.ds(b0, B_W)], idx_v)
        pltpu.async_copy(table_hbm.at[idx_v], rows_v, sem).wait()

        def g_body(g, carry2):
            gh = g // 8
            off = (g % 8) * 16
            tok_idx = g * 16 + lane
            for d in range(DIM):
                dvec = jnp.full((16,), d, jnp.int32)
                val = plsc.load_gather(rows_v, [tok_idx, dvec])
                trans_v[d // 8, gh, pl.ds((d % 8) * 128 + off, 16)] = val
            return carry2

        lax.fori_loop(0, B_W // 16, g_body, 0)
        pltpu.sync_copy(trans_v, out_hbm.at[s, :, pl.ds(bh0, BH_W), :])
        return carry

    lax.fori_loop(0, S, s_body, 0)


def kernel(token_ids, weight):
    tok_t = token_ids.T
    out5 = _gather_kernel(tok_t, weight)
    out = (
        out5.reshape(S, DIM // 8, B // 128, 8, 128)
        .transpose(2, 4, 0, 1, 3)
        .reshape(B, S, DIM)
    )
    return out
